# Initial kernel scaffold; baseline (speedup 1.0000x reference)
import jax
import jax.numpy as jnp
from jax.experimental import pallas as pl


def kernel(x, edge_index, edge_attr, batch, ew11, eb11, ew12, eb12, root1, cb1, bn1g, bn1b, ew21, eb21, ew22, eb22, root2, cb2, bn2g, bn2b, ew31, eb31, ew32, eb32, root3, cb3, bn3g, bn3b, gw1, gb1, gw2, gb2, fw1, fb1, n1g, n1b, fw2, fb2, n2g, n2b, fw3, fb3):
    raise NotImplementedError("write your pallas kernel here")


# speedup vs baseline: 1.2604x; 1.2604x over previous
import functools

import jax
import jax.numpy as jnp
from jax import lax
from jax.experimental import pallas as pl
from jax.experimental.pallas import tpu as pltpu
from jax.experimental.pallas import tpu_sc as plsc

N = 10000
E = 160000
FIN = 128
H = 8
ED = 16
G = 256
EPS = 1e-5

NC = 2
NS = 16
NW = NC * NS
NPAD = 10240
EP = 163840
EPT = EP // NW
CH = 128
NCHUNK = EPT // CH

_HI = jax.lax.Precision.HIGHEST
_SC_PARAMS = pltpu.CompilerParams(needs_layout_passes=False,
                                  use_tc_tiling_on_sc=False)


def _b16(x):
    return x.astype(jnp.bfloat16).astype(jnp.float32)


def _mmx(a, b):
    return jnp.dot(_b16(a), _b16(b), precision=_HI,
                   preferred_element_type=jnp.float32)


def _pre_body(ea_ref, wh_ref, bh_ref, h1_ref, h2_ref, h3_ref):
    t = jnp.maximum(_mmx(ea_ref[...], wh_ref[...]) + bh_ref[...], 0.0)
    h1_ref[...] = t[:, 0:8]
    h2_ref[...] = t[:, 8:16]
    h3_ref[...] = t[:, 16:24]


def _precompute(ea_p, wh, bh):
    eb = EP // 80
    return pl.pallas_call(
        _pre_body,
        grid=(80,),
        in_specs=[
            pl.BlockSpec((eb, ED), lambda i: (i, 0)),
            pl.BlockSpec((ED, 24), lambda i: (0, 0)),
            pl.BlockSpec((1, 24), lambda i: (0, 0)),
        ],
        out_specs=[
            pl.BlockSpec((eb, H), lambda i: (i, 0)),
            pl.BlockSpec((eb, H), lambda i: (i, 0)),
            pl.BlockSpec((eb, H), lambda i: (i, 0)),
        ],
        out_shape=[
            jax.ShapeDtypeStruct((EP, H), jnp.float32),
            jax.ShapeDtypeStruct((EP, H), jnp.float32),
            jax.ShapeDtypeStruct((EP, H), jnp.float32),
        ],
    )(ea_p, wh, bh)


def _gather_body(x_hbm, src_hbm, xg_hbm, idx, buf, sem):
    cid = lax.axis_index("c")
    sid = lax.axis_index("s")
    wid = sid * NC + cid

    def chunk_body(c, _):
        base = wid * EPT + c * CH
        pltpu.sync_copy(src_hbm.at[pl.ds(base, CH)], idx)
        pltpu.async_copy(x_hbm.at[idx], buf, sem).wait()
        pltpu.sync_copy(buf, xg_hbm.at[pl.ds(base, CH)])
        return 0
    lax.fori_loop(0, NCHUNK, chunk_body, 0)


@functools.lru_cache(maxsize=None)
def _make_sc_gather(width):
    return pl.kernel(
        _gather_body,
        out_type=jax.ShapeDtypeStruct((EP, width), jnp.float32),
        mesh=plsc.VectorSubcoreMesh(core_axis_name="c", subcore_axis_name="s",
                                    num_cores=NC, num_subcores=NS),
        scratch_types=[
            pltpu.VMEM((CH,), jnp.int32),
            pltpu.VMEM((CH, width), jnp.float32),
            pltpu.SemaphoreType.DMA,
        ],
        compiler_params=_SC_PARAMS,
    )


def _sc_gather(x_tab, src_p):
    return _make_sc_gather(x_tab.shape[1])(x_tab, src_p)


def _edge_body(xg_ref, h_ref, w2_ref, b2_ref, msg_ref, *, cin, blk):
    W = _mmx(h_ref[...], w2_ref[...]) + b2_ref[...]
    Wb = _b16(W)
    xgb = _b16(xg_ref[...])
    tmp = jnp.concatenate([xgb] * 8, axis=1)
    P = tmp * Wb
    cols = [jnp.sum(P[:, o * cin:(o + 1) * cin], axis=1, keepdims=True)
            for o in range(8)]
    msg_ref[...] = jnp.concatenate(
        cols + [jnp.ones((blk, 1), jnp.float32),
                jnp.zeros((blk, 7), jnp.float32)], axis=1)


def _edge_msg(xg, h, w2p, b2p, cin, blk):
    grid = EP // blk
    return pl.pallas_call(
        functools.partial(_edge_body, cin=cin, blk=blk),
        grid=(grid,),
        in_specs=[
            pl.BlockSpec((blk, cin), lambda i: (i, 0)),
            pl.BlockSpec((blk, H), lambda i: (i, 0)),
            pl.BlockSpec((H, 8 * cin), lambda i: (0, 0)),
            pl.BlockSpec((1, 8 * cin), lambda i: (0, 0)),
        ],
        out_specs=pl.BlockSpec((blk, 16), lambda i: (i, 0)),
        out_shape=jax.ShapeDtypeStruct((EP, 16), jnp.float32),
    )(xg, h, w2p, b2p)


def _scatter_body(msg_hbm, dst_hbm, part_hbm, idxd, msgbuf, zbuf, acc, sem):
    cid = lax.axis_index("c")
    sid = lax.axis_index("s")
    wid = sid * NC + cid

    zero16 = jnp.zeros((16,), jnp.float32)

    def _init(i, _):
        zbuf[i, :] = zero16
        return 0
    lax.fori_loop(0, CH, _init, 0)
    for j in range(NPAD // NS // CH):
        pltpu.sync_copy(zbuf, acc.at[pl.ds(sid * (NPAD // NS) + j * CH, CH)])
    plsc.subcore_barrier()

    def chunk_body(c, _):
        base = wid * EPT + c * CH
        pltpu.sync_copy(dst_hbm.at[pl.ds(base, CH)], idxd)
        pltpu.sync_copy(msg_hbm.at[pl.ds(base, CH)], msgbuf)
        pltpu.sync_copy(msgbuf, acc.at[idxd], add=True)
        return 0
    lax.fori_loop(0, NCHUNK, chunk_body, 0)

    plsc.subcore_barrier()
    for j in range(NPAD // NS // CH):
        row0 = sid * (NPAD // NS) + j * CH
        pltpu.sync_copy(acc.at[pl.ds(row0, CH)],
                        part_hbm.at[pl.ds(cid * NPAD + row0, CH)])


@functools.lru_cache(maxsize=1)
def _make_sc_scatter():
    return pl.kernel(
        _scatter_body,
        out_type=jax.ShapeDtypeStruct((NC * NPAD, 16), jnp.float32),
        mesh=plsc.VectorSubcoreMesh(core_axis_name="c", subcore_axis_name="s",
                                    num_cores=NC, num_subcores=NS),
        scratch_types=[
            pltpu.VMEM((CH,), jnp.int32),
            pltpu.VMEM((CH, 16), jnp.float32),
            pltpu.VMEM((CH, 16), jnp.float32),
            pltpu.VMEM_SHARED((NPAD, 16), jnp.float32),
            pltpu.SemaphoreType.DMA,
        ],
        compiler_params=_SC_PARAMS,
    )


def _sc_scatter(msg, dst_p):
    return _make_sc_scatter()(msg, dst_p)


_NBLK = 8
_NB = NPAD // _NBLK


def _agg_blk(p0_ref, p1_ref, xprev_ref, root_ref, cb_ref, mask_ref):
    p0 = p0_ref[...]
    p1 = p1_ref[...]
    S = p0[:, 0:8] + p1[:, 0:8]
    cnt = p0[:, 8:9] + p1[:, 8:9]
    agg = S / jnp.maximum(cnt, 1.0) + _mmx(xprev_ref[...], root_ref[...]) \
        + cb_ref[...]
    return agg * mask_ref[...]


def _bn_phases(agg, g_ref, b_ref, mask_ref, ssum, ssq):
    p = pl.program_id(0)
    i = pl.program_id(1)

    @pl.when(jnp.logical_and(p == 0, i == 0))
    def _():
        ssum[...] = jnp.zeros((1, H), jnp.float32)
        ssq[...] = jnp.zeros((1, H), jnp.float32)

    @pl.when(p == 0)
    def _():
        ssum[...] += jnp.sum(agg, axis=0, keepdims=True)
        ssq[...] += jnp.sum(agg * agg, axis=0, keepdims=True)

    m = ssum[...] / N
    v = ssq[...] / N - m * m
    return jnp.maximum((agg - m) * jax.lax.rsqrt(v + EPS) * g_ref[...]
                       + b_ref[...], 0.0) * mask_ref[...]


def _node_body(p0_ref, p1_ref, xprev_ref, root_ref, cb_ref, g_ref, b_ref,
               mask_ref, x_out_ref, ssum, ssq, *, resid):
    agg = _agg_blk(p0_ref, p1_ref, xprev_ref, root_ref, cb_ref, mask_ref)
    xn = _bn_phases(agg, g_ref, b_ref, mask_ref, ssum, ssq)

    @pl.when(pl.program_id(0) == 1)
    def _():
        x_out_ref[...] = xn + xprev_ref[...] if resid else xn


def _node_update(part, xprev, root, cb, g, b, mask, resid):
    cin = xprev.shape[1]
    full = lambda s: pl.BlockSpec(s, lambda p, i: (0, 0))
    return pl.pallas_call(
        functools.partial(_node_body, resid=resid),
        grid=(2, _NBLK),
        in_specs=[
            pl.BlockSpec((_NB, 16), lambda p, i: (i, 0)),
            pl.BlockSpec((_NB, 16), lambda p, i: (i, 0)),
            pl.BlockSpec((_NB, cin), lambda p, i: (i, 0)),
            full((cin, H)), full((1, H)), full((1, H)), full((1, H)),
            pl.BlockSpec((_NB, 1), lambda p, i: (i, 0)),
        ],
        out_specs=pl.BlockSpec((_NB, H), lambda p, i: (i, 0)),
        out_shape=jax.ShapeDtypeStruct((NPAD, H), jnp.float32),
        scratch_shapes=[
            pltpu.VMEM((1, H), jnp.float32),
            pltpu.VMEM((1, H), jnp.float32),
        ],
    )(part[0:NPAD], part[NPAD:2 * NPAD], xprev, root, cb, g, b, mask)


def _node3_body(p0_ref, p1_ref, xprev_ref, root_ref, cb_ref, g_ref, b_ref,
                gw1_ref, gb1_ref, gw2_ref, gb2_ref, mask_ref,
                x_out_ref, gate_out_ref, ssum, ssq):
    agg = _agg_blk(p0_ref, p1_ref, xprev_ref, root_ref, cb_ref, mask_ref)
    xn = _bn_phases(agg, g_ref, b_ref, mask_ref, ssum, ssq)

    @pl.when(pl.program_id(0) == 1)
    def _():
        x3 = xn + xprev_ref[...]
        x_out_ref[...] = x3
        gh = jnp.maximum(_mmx(x3, gw1_ref[...]) + gb1_ref[...], 0.0)
        gate_out_ref[...] = _mmx(gh, gw2_ref[...]) + gb2_ref[...]


def _node3(part, xprev, root, cb, g, b, mask, gw1, gb1, gw2, gb2):
    full = lambda s: pl.BlockSpec(s, lambda p, i: (0, 0))
    return pl.pallas_call(
        _node3_body,
        grid=(2, _NBLK),
        in_specs=[
            pl.BlockSpec((_NB, 16), lambda p, i: (i, 0)),
            pl.BlockSpec((_NB, 16), lambda p, i: (i, 0)),
            pl.BlockSpec((_NB, H), lambda p, i: (i, 0)),
            full((H, H)), full((1, H)), full((1, H)), full((1, H)),
            full((H, H // 2)), full((1, H // 2)), full((H // 2, 1)),
            full((1, 1)),
            pl.BlockSpec((_NB, 1), lambda p, i: (i, 0)),
        ],
        out_specs=[
            pl.BlockSpec((_NB, H), lambda p, i: (i, 0)),
            pl.BlockSpec((_NB, 1), lambda p, i: (i, 0)),
        ],
        out_shape=[
            jax.ShapeDtypeStruct((NPAD, H), jnp.float32),
            jax.ShapeDtypeStruct((NPAD, 1), jnp.float32),
        ],
        scratch_shapes=[
            pltpu.VMEM((1, H), jnp.float32),
            pltpu.VMEM((1, H), jnp.float32),
        ],
    )(part[0:NPAD], part[NPAD:2 * NPAD], xprev, root, cb, g, b,
      gw1, gb1, gw2, gb2, mask)


def _pool_body(x3_ref, gate_ref, batch_ref,
               fw1_ref, fb1_ref, n1g_ref, n1b_ref,
               fw2_ref, fb2_ref, n2g_ref, n2b_ref,
               fw3_ref, fb3_ref, out_ref, mx_acc, y_acc):
    p = pl.program_id(0)
    i = pl.program_id(1)
    batch = batch_ref[...]
    gi = lax.broadcasted_iota(jnp.int32, (G, _NB), 0)
    Mgn = (gi == batch).astype(jnp.float32)
    gate_row = gate_ref[...].reshape(1, _NB)

    @pl.when(jnp.logical_and(p == 0, i == 0))
    def _():
        mx_acc[...] = jnp.full((G, 1), -1e30, jnp.float32)

    @pl.when(p == 0)
    def _():
        blk_mx = jnp.max(jnp.where(Mgn > 0, gate_row, -1e30), axis=1,
                         keepdims=True)
        mx_acc[...] = jnp.maximum(mx_acc[...], blk_mx)

    @pl.when(jnp.logical_and(p == 1, i == 0))
    def _():
        y_acc[...] = jnp.zeros((G, 16), jnp.float32)

    @pl.when(p == 1)
    def _():
        gi2 = lax.broadcasted_iota(jnp.int32, (_NB, G), 1)
        Mng = (gi2 == batch.reshape(_NB, 1)).astype(jnp.float32)
        mxb = jnp.dot(Mng, mx_acc[...], precision=_HI,
                      preferred_element_type=jnp.float32)
        ex = jnp.exp(gate_ref[...] - mxb)
        Zb = jnp.concatenate(
            [x3_ref[...] * ex, ex, jnp.zeros((_NB, 7), jnp.float32)], axis=1)
        y_acc[...] += jnp.dot(Mgn, Zb, precision=_HI,
                              preferred_element_type=jnp.float32)

    @pl.when(jnp.logical_and(p == 1, i == _NBLK - 1))
    def _():
        Y = y_acc[...]
        den = Y[:, 8:9]
        pooled = Y[:, 0:8] * jnp.where(den > 0, 1.0 / den, 0.0)
        p1 = _mmx(pooled, fw1_ref[...]) + fb1_ref[...]
        m1 = jnp.mean(p1, axis=0, keepdims=True)
        v1 = jnp.mean(p1 * p1, axis=0, keepdims=True) - m1 * m1
        p1 = jnp.maximum((p1 - m1) * jax.lax.rsqrt(v1 + EPS) * n1g_ref[...]
                         + n1b_ref[...], 0.0)
        p2 = _mmx(p1, fw2_ref[...]) + fb2_ref[...]
        m2 = jnp.mean(p2, axis=0, keepdims=True)
        v2 = jnp.mean(p2 * p2, axis=0, keepdims=True) - m2 * m2
        p2 = jnp.maximum((p2 - m2) * jax.lax.rsqrt(v2 + EPS) * n2g_ref[...]
                         + n2b_ref[...], 0.0)
        out = _mmx(p2, fw3_ref[...]) + fb3_ref[...]
        out_ref[...] = out.reshape(1, G)


def _pool_head(x3, gate, batch2d, fw1, fb1, n1g, n1b,
               fw2, fb2, n2g, n2b, fw3, fb3):
    full = lambda s: pl.BlockSpec(s, lambda p, i: (0, 0))
    return pl.pallas_call(
        _pool_body,
        grid=(2, _NBLK),
        in_specs=[
            pl.BlockSpec((_NB, H), lambda p, i: (i, 0)),
            pl.BlockSpec((_NB, 1), lambda p, i: (i, 0)),
            pl.BlockSpec((1, _NB), lambda p, i: (0, i)),
            full((H, 2 * H)), full((1, 2 * H)), full((1, 2 * H)),
            full((1, 2 * H)),
            full((2 * H, H)), full((1, H)), full((1, H)), full((1, H)),
            full((H, 1)), full((1, 1)),
        ],
        out_specs=pl.BlockSpec((1, G), lambda p, i: (0, 0)),
        out_shape=jax.ShapeDtypeStruct((1, G), jnp.float32),
        scratch_shapes=[
            pltpu.VMEM((G, 1), jnp.float32),
            pltpu.VMEM((G, 16), jnp.float32),
        ],
    )(x3, gate, batch2d, fw1, fb1, n1g, n1b, fw2, fb2, n2g, n2b, fw3, fb3)


def _omajor(ew, eb, cin):
    wp = ew.reshape(H, cin, H).transpose(0, 2, 1).reshape(H, cin * H)
    bp = eb.reshape(cin, H).T.reshape(1, cin * H)
    return wp, bp


def kernel(x, edge_index, edge_attr, batch,
           ew11, eb11, ew12, eb12, root1, cb1, bn1g, bn1b,
           ew21, eb21, ew22, eb22, root2, cb2, bn2g, bn2b,
           ew31, eb31, ew32, eb32, root3, cb3, bn3g, bn3b,
           gw1, gb1, gw2, gb2,
           fw1, fb1, n1g, n1b, fw2, fb2, n2g, n2b, fw3, fb3):
    f32 = jnp.float32
    src_p = jnp.pad(edge_index[0], (0, EP - E))
    dst_p = jnp.pad(edge_index[1], (0, EP - E), constant_values=N)
    ea_p = jnp.pad(edge_attr, ((0, EP - E), (0, 0)))
    x_p = jnp.pad(x, ((0, NPAD - N), (0, 0)))
    batch_p = jnp.pad(batch, (0, NPAD - N), constant_values=G + 44)

    wh = jnp.concatenate([ew11, ew21, ew31], axis=1)
    bh = jnp.concatenate([eb11, eb21, eb31]).reshape(1, 24)
    w12p, b12p = _omajor(ew12, eb12, FIN)
    w22p, b22p = _omajor(ew22, eb22, H)
    w32p, b32p = _omajor(ew32, eb32, H)
    mask = (jnp.arange(NPAD) < N).astype(f32).reshape(NPAD, 1)

    h1, h2, h3 = _precompute(ea_p, wh, bh)

    xg1 = _sc_gather(x_p, src_p)
    msg1 = _edge_msg(xg1, h1, w12p, b12p, FIN, 512)
    part1 = _sc_scatter(msg1, dst_p)
    x1 = _node_update(part1, x_p, root1, cb1.reshape(1, H),
                      bn1g.reshape(1, H), bn1b.reshape(1, H), mask,
                      resid=False)

    xg2 = _sc_gather(x1, src_p)
    msg2 = _edge_msg(xg2, h2, w22p, b22p, H, 2048)
    part2 = _sc_scatter(msg2, dst_p)
    x2 = _node_update(part2, x1, root2, cb2.reshape(1, H),
                      bn2g.reshape(1, H), bn2b.reshape(1, H), mask,
                      resid=True)

    xg3 = _sc_gather(x2, src_p)
    msg3 = _edge_msg(xg3, h3, w32p, b32p, H, 2048)
    part3 = _sc_scatter(msg3, dst_p)
    x3, gate = _node3(part3, x2, root3, cb3.reshape(1, H),
                      bn3g.reshape(1, H), bn3b.reshape(1, H), mask,
                      gw1, gb1.reshape(1, H // 2), gw2, gb2.reshape(1, 1))

    out = _pool_head(x3, gate, batch_p.reshape(1, NPAD),
                     fw1, fb1.reshape(1, 2 * H), n1g.reshape(1, 2 * H),
                     n1b.reshape(1, 2 * H),
                     fw2, fb2.reshape(1, H), n2g.reshape(1, H),
                     n2b.reshape(1, H),
                     fw3, fb3.reshape(1, 1))
    return out.reshape(G)


# speedup vs baseline: 1.3839x; 1.0979x over previous
import functools

import jax
import jax.numpy as jnp
from jax import lax
from jax.experimental import pallas as pl
from jax.experimental.pallas import tpu as pltpu
from jax.experimental.pallas import tpu_sc as plsc

N = 10000
E = 160000
FIN = 128
H = 8
ED = 16
G = 256
EPS = 1e-5

NC = 2
NS = 16
NW = NC * NS
NPAD = 10240
EP = 163840
EPT = EP // NW
CH = 128
NCHUNK = EPT // CH

_HI = jax.lax.Precision.HIGHEST
_SC_PARAMS = pltpu.CompilerParams(needs_layout_passes=False,
                                  use_tc_tiling_on_sc=False)


def _b16(x):
    return x.astype(jnp.bfloat16).astype(jnp.float32)


def _mmx(a, b):
    return jnp.dot(_b16(a), _b16(b), precision=_HI,
                   preferred_element_type=jnp.float32)


def _pre_body(ea_ref, wh_ref, bh_ref, h1_ref, h2_ref, h3_ref):
    t = jnp.maximum(_mmx(ea_ref[...], wh_ref[...]) + bh_ref[...], 0.0)
    h1_ref[...] = t[:, 0:8]
    h2_ref[...] = t[:, 8:16]
    h3_ref[...] = t[:, 16:24]


def _precompute(ea_p, wh, bh):
    eb = EP // 80
    return pl.pallas_call(
        _pre_body,
        grid=(80,),
        in_specs=[
            pl.BlockSpec((eb, ED), lambda i: (i, 0)),
            pl.BlockSpec((ED, 24), lambda i: (0, 0)),
            pl.BlockSpec((1, 24), lambda i: (0, 0)),
        ],
        out_specs=[
            pl.BlockSpec((eb, H), lambda i: (i, 0)),
            pl.BlockSpec((eb, H), lambda i: (i, 0)),
            pl.BlockSpec((eb, H), lambda i: (i, 0)),
        ],
        out_shape=[
            jax.ShapeDtypeStruct((EP, H), jnp.float32),
            jax.ShapeDtypeStruct((EP, H), jnp.float32),
            jax.ShapeDtypeStruct((EP, H), jnp.float32),
        ],
    )(ea_p, wh, bh)


def _gather_body(x_hbm, src2_hbm, xg_hbm, idx0, idx1, buf0, buf1,
                 sem0, sem1, *, width, sch):
    cid = lax.axis_index("c")
    sid = lax.axis_index("s")
    wid = sid * NC + cid
    k = sch // CH
    nsc = EPT // sch
    idxs = (idx0, idx1)
    bufs = (buf0, buf1)
    sems = (sem0, sem1)

    def fire(t, b):
        row0 = (wid * EPT + t * sch) // CH
        pltpu.sync_copy(src2_hbm.at[pl.ds(row0, k)], idxs[b])
        for j in range(k):
            pltpu.async_copy(x_hbm.at[idxs[b].at[j]],
                             bufs[b].at[pl.ds(j * CH, CH)], sems[b])

    def drain(b):
        for j in range(k):
            pltpu.make_async_copy(x_hbm.at[idxs[b].at[j]],
                                  bufs[b].at[pl.ds(j * CH, CH)],
                                  sems[b]).wait()

    fire(0, 0)
    for t in range(nsc):
        b = t % 2
        if t + 1 < nsc:
            fire(t + 1, (t + 1) % 2)
        drain(b)
        pltpu.sync_copy(bufs[b], xg_hbm.at[pl.ds(wid * EPT + t * sch, sch)])


@functools.lru_cache(maxsize=None)
def _make_sc_gather(width):
    sch = 256 if width > 16 else 1024
    return pl.kernel(
        functools.partial(_gather_body, width=width, sch=sch),
        out_type=jax.ShapeDtypeStruct((EP, width), jnp.float32),
        mesh=plsc.VectorSubcoreMesh(core_axis_name="c", subcore_axis_name="s",
                                    num_cores=NC, num_subcores=NS),
        scratch_types=[
            pltpu.VMEM((sch // CH, CH), jnp.int32),
            pltpu.VMEM((sch // CH, CH), jnp.int32),
            pltpu.VMEM((sch, width), jnp.float32),
            pltpu.VMEM((sch, width), jnp.float32),
            pltpu.SemaphoreType.DMA,
            pltpu.SemaphoreType.DMA,
        ],
        compiler_params=_SC_PARAMS,
    )


def _sc_gather(x_tab, src2):
    return _make_sc_gather(x_tab.shape[1])(x_tab, src2)


def _edge_body(xg_ref, h_ref, w2_ref, b2_ref, msg_ref, *, cin, blk):
    W = _mmx(h_ref[...], w2_ref[...]) + b2_ref[...]
    Wb = _b16(W)
    xgb = _b16(xg_ref[...])
    tmp = jnp.concatenate([xgb] * 8, axis=1)
    P = tmp * Wb
    cols = [jnp.sum(P[:, o * cin:(o + 1) * cin], axis=1, keepdims=True)
            for o in range(8)]
    msg_ref[...] = jnp.concatenate(
        cols + [jnp.ones((blk, 1), jnp.float32),
                jnp.zeros((blk, 7), jnp.float32)], axis=1)


def _edge_msg(xg, h, w2p, b2p, cin, blk):
    grid = EP // blk
    return pl.pallas_call(
        functools.partial(_edge_body, cin=cin, blk=blk),
        grid=(grid,),
        in_specs=[
            pl.BlockSpec((blk, cin), lambda i: (i, 0)),
            pl.BlockSpec((blk, H), lambda i: (i, 0)),
            pl.BlockSpec((H, 8 * cin), lambda i: (0, 0)),
            pl.BlockSpec((1, 8 * cin), lambda i: (0, 0)),
        ],
        out_specs=pl.BlockSpec((blk, 16), lambda i: (i, 0)),
        out_shape=jax.ShapeDtypeStruct((EP, 16), jnp.float32),
    )(xg, h, w2p, b2p)


_SSCH = 1024
_SK = _SSCH // CH
_SNSC = EPT // _SSCH
_RPS = NPAD // NS


def _scatter_body(msg_hbm, dst2_hbm, part_hbm, idxd0, idxd1, msgb0, msgb1,
                  zbuf, acc, lsem, ssem):
    cid = lax.axis_index("c")
    sid = lax.axis_index("s")
    wid = sid * NC + cid
    idxs = (idxd0, idxd1)
    msgs = (msgb0, msgb1)

    zero16 = jnp.zeros((16,), jnp.float32)

    def _init(i, _):
        zbuf[i, :] = zero16
        return 0
    lax.fori_loop(0, _RPS, _init, 0)
    pltpu.sync_copy(zbuf, acc.at[pl.ds(sid * _RPS, _RPS)])
    plsc.subcore_barrier()

    def fire_loads(t, b):
        base = wid * EPT + t * _SSCH
        pltpu.async_copy(dst2_hbm.at[pl.ds(base // CH, _SK)], idxs[b], lsem)
        pltpu.async_copy(msg_hbm.at[pl.ds(base, _SSCH)], msgs[b], lsem)

    def drain_loads(t, b):
        base = wid * EPT + t * _SSCH
        pltpu.make_async_copy(dst2_hbm.at[pl.ds(base // CH, _SK)],
                              idxs[b], lsem).wait()
        pltpu.make_async_copy(msg_hbm.at[pl.ds(base, _SSCH)],
                              msgs[b], lsem).wait()

    fire_loads(0, 0)
    for t in range(_SNSC):
        b = t % 2
        drain_loads(t, b)
        if t + 1 < _SNSC:
            fire_loads(t + 1, (t + 1) % 2)
        for j in range(_SK):
            pltpu.async_copy(msgs[b].at[pl.ds(j * CH, CH)],
                             acc.at[idxs[b].at[j]], ssem, add=True)
        for j in range(_SK):
            pltpu.make_async_copy(msgs[b].at[pl.ds(j * CH, CH)],
                                  acc.at[idxs[b].at[j]], ssem).wait()

    plsc.subcore_barrier()
    pltpu.sync_copy(acc.at[pl.ds(sid * _RPS, _RPS)],
                    part_hbm.at[pl.ds(cid * NPAD + sid * _RPS, _RPS)])


@functools.lru_cache(maxsize=1)
def _make_sc_scatter():
    return pl.kernel(
        _scatter_body,
        out_type=jax.ShapeDtypeStruct((NC * NPAD, 16), jnp.float32),
        mesh=plsc.VectorSubcoreMesh(core_axis_name="c", subcore_axis_name="s",
                                    num_cores=NC, num_subcores=NS),
        scratch_types=[
            pltpu.VMEM((_SK, CH), jnp.int32),
            pltpu.VMEM((_SK, CH), jnp.int32),
            pltpu.VMEM((_SSCH, 16), jnp.float32),
            pltpu.VMEM((_SSCH, 16), jnp.float32),
            pltpu.VMEM((_RPS, 16), jnp.float32),
            pltpu.VMEM_SHARED((NPAD, 16), jnp.float32),
            pltpu.SemaphoreType.DMA,
            pltpu.SemaphoreType.DMA,
        ],
        compiler_params=_SC_PARAMS,
    )


def _sc_scatter(msg, dst2):
    return _make_sc_scatter()(msg, dst2)


_NBLK = 8
_NB = NPAD // _NBLK


def _agg_blk(p0_ref, p1_ref, xprev_ref, root_ref, cb_ref, mask_ref):
    p0 = p0_ref[...]
    p1 = p1_ref[...]
    S = p0[:, 0:8] + p1[:, 0:8]
    cnt = p0[:, 8:9] + p1[:, 8:9]
    agg = S / jnp.maximum(cnt, 1.0) + _mmx(xprev_ref[...], root_ref[...]) \
        + cb_ref[...]
    return agg * mask_ref[...]


def _bn_phases(agg, g_ref, b_ref, mask_ref, ssum, ssq):
    p = pl.program_id(0)
    i = pl.program_id(1)

    @pl.when(jnp.logical_and(p == 0, i == 0))
    def _():
        ssum[...] = jnp.zeros((1, H), jnp.float32)
        ssq[...] = jnp.zeros((1, H), jnp.float32)

    @pl.when(p == 0)
    def _():
        ssum[...] += jnp.sum(agg, axis=0, keepdims=True)
        ssq[...] += jnp.sum(agg * agg, axis=0, keepdims=True)

    m = ssum[...] / N
    v = ssq[...] / N - m * m
    return jnp.maximum((agg - m) * jax.lax.rsqrt(v + EPS) * g_ref[...]
                       + b_ref[...], 0.0) * mask_ref[...]


def _node_body(p0_ref, p1_ref, xprev_ref, root_ref, cb_ref, g_ref, b_ref,
               mask_ref, x_out_ref, ssum, ssq, *, resid):
    agg = _agg_blk(p0_ref, p1_ref, xprev_ref, root_ref, cb_ref, mask_ref)
    xn = _bn_phases(agg, g_ref, b_ref, mask_ref, ssum, ssq)

    @pl.when(pl.program_id(0) == 1)
    def _():
        x_out_ref[...] = xn + xprev_ref[...] if resid else xn


def _node_update(part, xprev, root, cb, g, b, mask, resid):
    cin = xprev.shape[1]
    full = lambda s: pl.BlockSpec(s, lambda p, i: (0, 0))
    return pl.pallas_call(
        functools.partial(_node_body, resid=resid),
        grid=(2, _NBLK),
        in_specs=[
            pl.BlockSpec((_NB, 16), lambda p, i: (i, 0)),
            pl.BlockSpec((_NB, 16), lambda p, i: (i, 0)),
            pl.BlockSpec((_NB, cin), lambda p, i: (i, 0)),
            full((cin, H)), full((1, H)), full((1, H)), full((1, H)),
            pl.BlockSpec((_NB, 1), lambda p, i: (i, 0)),
        ],
        out_specs=pl.BlockSpec((_NB, H), lambda p, i: (i, 0)),
        out_shape=jax.ShapeDtypeStruct((NPAD, H), jnp.float32),
        scratch_shapes=[
            pltpu.VMEM((1, H), jnp.float32),
            pltpu.VMEM((1, H), jnp.float32),
        ],
    )(part[0:NPAD], part[NPAD:2 * NPAD], xprev, root, cb, g, b, mask)


def _node3_body(p0_ref, p1_ref, xprev_ref, root_ref, cb_ref, g_ref, b_ref,
                gw1_ref, gb1_ref, gw2_ref, gb2_ref, mask_ref,
                x_out_ref, gate_out_ref, ssum, ssq):
    agg = _agg_blk(p0_ref, p1_ref, xprev_ref, root_ref, cb_ref, mask_ref)
    xn = _bn_phases(agg, g_ref, b_ref, mask_ref, ssum, ssq)

    @pl.when(pl.program_id(0) == 1)
    def _():
        x3 = xn + xprev_ref[...]
        x_out_ref[...] = x3
        gh = jnp.maximum(_mmx(x3, gw1_ref[...]) + gb1_ref[...], 0.0)
        gate_out_ref[...] = _mmx(gh, gw2_ref[...]) + gb2_ref[...]


def _node3(part, xprev, root, cb, g, b, mask, gw1, gb1, gw2, gb2):
    full = lambda s: pl.BlockSpec(s, lambda p, i: (0, 0))
    return pl.pallas_call(
        _node3_body,
        grid=(2, _NBLK),
        in_specs=[
            pl.BlockSpec((_NB, 16), lambda p, i: (i, 0)),
            pl.BlockSpec((_NB, 16), lambda p, i: (i, 0)),
            pl.BlockSpec((_NB, H), lambda p, i: (i, 0)),
            full((H, H)), full((1, H)), full((1, H)), full((1, H)),
            full((H, H // 2)), full((1, H // 2)), full((H // 2, 1)),
            full((1, 1)),
            pl.BlockSpec((_NB, 1), lambda p, i: (i, 0)),
        ],
        out_specs=[
            pl.BlockSpec((_NB, H), lambda p, i: (i, 0)),
            pl.BlockSpec((_NB, 1), lambda p, i: (i, 0)),
        ],
        out_shape=[
            jax.ShapeDtypeStruct((NPAD, H), jnp.float32),
            jax.ShapeDtypeStruct((NPAD, 1), jnp.float32),
        ],
        scratch_shapes=[
            pltpu.VMEM((1, H), jnp.float32),
            pltpu.VMEM((1, H), jnp.float32),
        ],
    )(part[0:NPAD], part[NPAD:2 * NPAD], xprev, root, cb, g, b,
      gw1, gb1, gw2, gb2, mask)


def _pool_body(x3_ref, gate_ref, batch_ref,
               fw1_ref, fb1_ref, n1g_ref, n1b_ref,
               fw2_ref, fb2_ref, n2g_ref, n2b_ref,
               fw3_ref, fb3_ref, out_ref, mx_acc, y_acc):
    p = pl.program_id(0)
    i = pl.program_id(1)
    batch = batch_ref[...]
    gi = lax.broadcasted_iota(jnp.int32, (G, _NB), 0)
    Mgn = (gi == batch).astype(jnp.float32)
    gate_row = gate_ref[...].reshape(1, _NB)

    @pl.when(jnp.logical_and(p == 0, i == 0))
    def _():
        mx_acc[...] = jnp.full((G, 1), -1e30, jnp.float32)

    @pl.when(p == 0)
    def _():
        blk_mx = jnp.max(jnp.where(Mgn > 0, gate_row, -1e30), axis=1,
                         keepdims=True)
        mx_acc[...] = jnp.maximum(mx_acc[...], blk_mx)

    @pl.when(jnp.logical_and(p == 1, i == 0))
    def _():
        y_acc[...] = jnp.zeros((G, 16), jnp.float32)

    @pl.when(p == 1)
    def _():
        gi2 = lax.broadcasted_iota(jnp.int32, (_NB, G), 1)
        Mng = (gi2 == batch.reshape(_NB, 1)).astype(jnp.float32)
        mxb = jnp.dot(Mng, mx_acc[...], precision=_HI,
                      preferred_element_type=jnp.float32)
        ex = jnp.exp(gate_ref[...] - mxb)
        Zb = jnp.concatenate(
            [x3_ref[...] * ex, ex, jnp.zeros((_NB, 7), jnp.float32)], axis=1)
        y_acc[...] += jnp.dot(Mgn, Zb, precision=_HI,
                              preferred_element_type=jnp.float32)

    @pl.when(jnp.logical_and(p == 1, i == _NBLK - 1))
    def _():
        Y = y_acc[...]
        den = Y[:, 8:9]
        pooled = Y[:, 0:8] * jnp.where(den > 0, 1.0 / den, 0.0)
        p1 = _mmx(pooled, fw1_ref[...]) + fb1_ref[...]
        m1 = jnp.mean(p1, axis=0, keepdims=True)
        v1 = jnp.mean(p1 * p1, axis=0, keepdims=True) - m1 * m1
        p1 = jnp.maximum((p1 - m1) * jax.lax.rsqrt(v1 + EPS) * n1g_ref[...]
                         + n1b_ref[...], 0.0)
        p2 = _mmx(p1, fw2_ref[...]) + fb2_ref[...]
        m2 = jnp.mean(p2, axis=0, keepdims=True)
        v2 = jnp.mean(p2 * p2, axis=0, keepdims=True) - m2 * m2
        p2 = jnp.maximum((p2 - m2) * jax.lax.rsqrt(v2 + EPS) * n2g_ref[...]
                         + n2b_ref[...], 0.0)
        out = _mmx(p2, fw3_ref[...]) + fb3_ref[...]
        out_ref[...] = out.reshape(1, G)


def _pool_head(x3, gate, batch2d, fw1, fb1, n1g, n1b,
               fw2, fb2, n2g, n2b, fw3, fb3):
    full = lambda s: pl.BlockSpec(s, lambda p, i: (0, 0))
    return pl.pallas_call(
        _pool_body,
        grid=(2, _NBLK),
        in_specs=[
            pl.BlockSpec((_NB, H), lambda p, i: (i, 0)),
            pl.BlockSpec((_NB, 1), lambda p, i: (i, 0)),
            pl.BlockSpec((1, _NB), lambda p, i: (0, i)),
            full((H, 2 * H)), full((1, 2 * H)), full((1, 2 * H)),
            full((1, 2 * H)),
            full((2 * H, H)), full((1, H)), full((1, H)), full((1, H)),
            full((H, 1)), full((1, 1)),
        ],
        out_specs=pl.BlockSpec((1, G), lambda p, i: (0, 0)),
        out_shape=jax.ShapeDtypeStruct((1, G), jnp.float32),
        scratch_shapes=[
            pltpu.VMEM((G, 1), jnp.float32),
            pltpu.VMEM((G, 16), jnp.float32),
        ],
    )(x3, gate, batch2d, fw1, fb1, n1g, n1b, fw2, fb2, n2g, n2b, fw3, fb3)


def _omajor(ew, eb, cin):
    wp = ew.reshape(H, cin, H).transpose(0, 2, 1).reshape(H, cin * H)
    bp = eb.reshape(cin, H).T.reshape(1, cin * H)
    return wp, bp


def kernel(x, edge_index, edge_attr, batch,
           ew11, eb11, ew12, eb12, root1, cb1, bn1g, bn1b,
           ew21, eb21, ew22, eb22, root2, cb2, bn2g, bn2b,
           ew31, eb31, ew32, eb32, root3, cb3, bn3g, bn3b,
           gw1, gb1, gw2, gb2,
           fw1, fb1, n1g, n1b, fw2, fb2, n2g, n2b, fw3, fb3):
    f32 = jnp.float32
    src2 = jnp.pad(edge_index[0], (0, EP - E)).reshape(EP // CH, CH)
    dst2 = jnp.pad(edge_index[1], (0, EP - E),
                   constant_values=N).reshape(EP // CH, CH)
    ea_p = jnp.pad(edge_attr, ((0, EP - E), (0, 0)))
    x_p = jnp.pad(x, ((0, NPAD - N), (0, 0)))
    batch_p = jnp.pad(batch, (0, NPAD - N), constant_values=G + 44)

    wh = jnp.concatenate([ew11, ew21, ew31], axis=1)
    bh = jnp.concatenate([eb11, eb21, eb31]).reshape(1, 24)
    w12p, b12p = _omajor(ew12, eb12, FIN)
    w22p, b22p = _omajor(ew22, eb22, H)
    w32p, b32p = _omajor(ew32, eb32, H)
    mask = (jnp.arange(NPAD) < N).astype(f32).reshape(NPAD, 1)

    h1, h2, h3 = _precompute(ea_p, wh, bh)

    xg1 = _sc_gather(x_p, src2)
    msg1 = _edge_msg(xg1, h1, w12p, b12p, FIN, 512)
    part1 = _sc_scatter(msg1, dst2)
    x1 = _node_update(part1, x_p, root1, cb1.reshape(1, H),
                      bn1g.reshape(1, H), bn1b.reshape(1, H), mask,
                      resid=False)

    xg2 = _sc_gather(x1, src2)
    msg2 = _edge_msg(xg2, h2, w22p, b22p, H, 2048)
    part2 = _sc_scatter(msg2, dst2)
    x2 = _node_update(part2, x1, root2, cb2.reshape(1, H),
                      bn2g.reshape(1, H), bn2b.reshape(1, H), mask,
                      resid=True)

    xg3 = _sc_gather(x2, src2)
    msg3 = _edge_msg(xg3, h3, w32p, b32p, H, 2048)
    part3 = _sc_scatter(msg3, dst2)
    x3, gate = _node3(part3, x2, root3, cb3.reshape(1, H),
                      bn3g.reshape(1, H), bn3b.reshape(1, H), mask,
                      gw1, gb1.reshape(1, H // 2), gw2, gb2.reshape(1, 1))

    out = _pool_head(x3, gate, batch_p.reshape(1, NPAD),
                     fw1, fb1.reshape(1, 2 * H), n1g.reshape(1, 2 * H),
                     n1b.reshape(1, 2 * H),
                     fw2, fb2.reshape(1, H), n2g.reshape(1, H),
                     n2b.reshape(1, H),
                     fw3, fb3.reshape(1, 1))
    return out.reshape(G)
